# Initial kernel scaffold; baseline (speedup 1.0000x reference)
import jax
import jax.numpy as jnp
from jax.experimental import pallas as pl


def kernel(hidden_states, W_lin, b_lin, gate_w, lefts, rights):
    raise NotImplementedError("write your pallas kernel here")


# speedup vs baseline: 3.4407x; 3.4407x over previous
import functools

import jax
import jax.numpy as jnp
from jax.experimental import pallas as pl

E = 8
K = 2
R = 16
ALPHA = 16
SCALING = ALPHA / R

TM = 512


def _fused_kernel(x_ref, wt_ref, b_ref, gt_ref, lf_ref, rf_ref, o_ref):
    x = x_ref[...]

    logits = jnp.dot(x, gt_ref[...], preferred_element_type=jnp.float32)
    mx = jnp.max(logits, axis=-1, keepdims=True)
    ex = jnp.exp(logits - mx)
    scores = ex / jnp.sum(ex, axis=-1, keepdims=True)

    idx = jax.lax.broadcasted_iota(jnp.int32, scores.shape, 1)
    m1 = jnp.max(scores, axis=-1, keepdims=True)
    i1 = jnp.min(jnp.where(scores == m1, idx, E), axis=-1, keepdims=True)
    sel1 = idx == i1
    masked = jnp.where(sel1, -jnp.inf, scores)
    m2 = jnp.max(masked, axis=-1, keepdims=True)
    i2 = jnp.min(jnp.where(masked == m2, idx, E), axis=-1, keepdims=True)
    comb = jnp.where(sel1 | (idx == i2), scores, 0.0)

    er = jax.lax.broadcasted_iota(jnp.int32, (E, E * R), 1) // R
    ee = jax.lax.broadcasted_iota(jnp.int32, (E, E * R), 0)
    expand = (er == ee).astype(jnp.float32)
    comb_wide = jnp.dot(comb, expand, preferred_element_type=jnp.float32)

    h = jnp.dot(x, lf_ref[...], preferred_element_type=jnp.float32)
    hw = h * comb_wide * SCALING

    out = jnp.dot(x, wt_ref[...], preferred_element_type=jnp.float32)
    out += jnp.dot(hw, rf_ref[...], preferred_element_type=jnp.float32)
    o_ref[...] = out + b_ref[...]


@functools.partial(jax.jit, static_argnames=())
def _run(flat, w_t, b2, g_t, lefts_flat, rights_flat):
    T, D = flat.shape
    grid = (T // TM,)
    return pl.pallas_call(
        _fused_kernel,
        grid=grid,
        in_specs=[
            pl.BlockSpec((TM, D), lambda i: (i, 0)),
            pl.BlockSpec((D, D), lambda i: (0, 0)),
            pl.BlockSpec((1, D), lambda i: (0, 0)),
            pl.BlockSpec((D, E), lambda i: (0, 0)),
            pl.BlockSpec((D, E * R), lambda i: (0, 0)),
            pl.BlockSpec((E * R, D), lambda i: (0, 0)),
        ],
        out_specs=pl.BlockSpec((TM, D), lambda i: (i, 0)),
        out_shape=jax.ShapeDtypeStruct((T, D), jnp.float32),
    )(flat, w_t, b2, g_t, lefts_flat, rights_flat)


def kernel(hidden_states, W_lin, b_lin, gate_w, lefts, rights):
    bsz, seq_len, dim = hidden_states.shape
    flat = hidden_states.reshape(-1, dim)
    w_t = W_lin.T
    g_t = gate_w.T
    d = lefts.shape[1]
    lefts_flat = lefts.transpose(1, 0, 2).reshape(d, E * R)
    rights_flat = rights.reshape(E * R, -1)
    b2 = b_lin.reshape(1, -1)
    out = _run(flat, w_t, b2, g_t, lefts_flat, rights_flat)
    return out.reshape(bsz, seq_len, -1)


# speedup vs baseline: 4.5793x; 1.3309x over previous
import functools

import jax
import jax.numpy as jnp
from jax.experimental import pallas as pl
from jax.experimental.pallas import tpu as pltpu

E = 8
K = 2
R = 16
ALPHA = 16
SCALING = ALPHA / R

TM = 512

_DN_T = (((1,), (1,)), ((), ()))


def _fused_kernel(x_ref, w_ref, b_ref, gt_ref, lf_ref, rf_ref, o_ref,
                  wb_ref, lfb_ref, rfb_ref):
    i = pl.program_id(0)

    @pl.when(i == 0)
    def _cast_weights():
        wb_ref[...] = w_ref[...].astype(jnp.bfloat16)
        lfb_ref[...] = lf_ref[...].astype(jnp.bfloat16)
        rfb_ref[...] = rf_ref[...].astype(jnp.bfloat16)

    x = x_ref[...]
    xb = x.astype(jnp.bfloat16)

    logits = jnp.dot(x, gt_ref[...], preferred_element_type=jnp.float32)
    mx = jnp.max(logits, axis=-1, keepdims=True)
    ex = jnp.exp(logits - mx)
    scores = ex / jnp.sum(ex, axis=-1, keepdims=True)

    idx = jax.lax.broadcasted_iota(jnp.int32, scores.shape, 1)
    m1 = jnp.max(scores, axis=-1, keepdims=True)
    i1 = jnp.min(jnp.where(scores == m1, idx, E), axis=-1, keepdims=True)
    sel1 = idx == i1
    masked = jnp.where(sel1, -jnp.inf, scores)
    m2 = jnp.max(masked, axis=-1, keepdims=True)
    i2 = jnp.min(jnp.where(masked == m2, idx, E), axis=-1, keepdims=True)
    comb = jnp.where(sel1 | (idx == i2), scores, 0.0)

    er = jax.lax.broadcasted_iota(jnp.int32, (E, E * R), 1) // R
    ee = jax.lax.broadcasted_iota(jnp.int32, (E, E * R), 0)
    expand = (er == ee).astype(jnp.float32)
    comb_wide = jnp.dot(comb, expand, preferred_element_type=jnp.float32)

    h = jnp.dot(xb, lfb_ref[...], preferred_element_type=jnp.float32)
    hw = (h * comb_wide * SCALING).astype(jnp.bfloat16)

    out = jax.lax.dot_general(xb, wb_ref[...], _DN_T,
                              preferred_element_type=jnp.float32)
    out += jnp.dot(hw, rfb_ref[...], preferred_element_type=jnp.float32)
    o_ref[...] = out + b_ref[...]


@jax.jit
def _run(flat, w, b2, g_t, lefts_flat, rights_flat):
    T, D = flat.shape
    grid = (T // TM,)
    return pl.pallas_call(
        _fused_kernel,
        grid=grid,
        in_specs=[
            pl.BlockSpec((TM, D), lambda i: (i, 0)),
            pl.BlockSpec((D, D), lambda i: (0, 0)),
            pl.BlockSpec((1, D), lambda i: (0, 0)),
            pl.BlockSpec((D, E), lambda i: (0, 0)),
            pl.BlockSpec((D, E * R), lambda i: (0, 0)),
            pl.BlockSpec((E * R, D), lambda i: (0, 0)),
        ],
        out_specs=pl.BlockSpec((TM, D), lambda i: (i, 0)),
        out_shape=jax.ShapeDtypeStruct((T, D), jnp.float32),
        scratch_shapes=[
            pltpu.VMEM((D, D), jnp.bfloat16),
            pltpu.VMEM((D, E * R), jnp.bfloat16),
            pltpu.VMEM((E * R, D), jnp.bfloat16),
        ],
    )(flat, w, b2, g_t, lefts_flat, rights_flat)


def kernel(hidden_states, W_lin, b_lin, gate_w, lefts, rights):
    bsz, seq_len, dim = hidden_states.shape
    flat = hidden_states.reshape(-1, dim)
    g_t = gate_w.T
    d = lefts.shape[1]
    lefts_flat = lefts.transpose(1, 0, 2).reshape(d, E * R)
    rights_flat = rights.reshape(E * R, -1)
    b2 = b_lin.reshape(1, -1)
    out = _run(flat, W_lin, b2, g_t, lefts_flat, rights_flat)
    return out.reshape(bsz, seq_len, -1)


# speedup vs baseline: 5.4136x; 1.1822x over previous
import functools

import jax
import jax.numpy as jnp
from jax.experimental import pallas as pl
from jax.experimental.pallas import tpu as pltpu

E = 8
K = 2
R = 16
ALPHA = 16
SCALING = ALPHA / R

TM = 512

_DN_T = (((1,), (1,)), ((), ()))
_DN_TG = (((0,), (1,)), ((), ()))
_DN_TL = (((0,), (0,)), ((), ()))


def _fused_kernel(x_ref, w_ref, b_ref, gt_ref, lf_ref, rf_ref, o_ref,
                  wb_ref, lfb_ref, rfb_ref):
    i = pl.program_id(0)

    @pl.when(i == 0)
    def _cast_weights():
        wb_ref[...] = w_ref[...].astype(jnp.bfloat16)
        lfb_ref[...] = lf_ref[...].astype(jnp.bfloat16)
        rfb_ref[...] = rf_ref[...].astype(jnp.bfloat16)

    x = x_ref[...]
    xb = x.astype(jnp.bfloat16)

    logits = jax.lax.dot_general(gt_ref[...], x, _DN_TG,
                                 preferred_element_type=jnp.float32)
    mx = jnp.max(logits, axis=0, keepdims=True)
    ex = jnp.exp(logits - mx)
    scores = ex / jnp.sum(ex, axis=0, keepdims=True)

    idx = jax.lax.broadcasted_iota(jnp.int32, scores.shape, 0)
    m1 = jnp.max(scores, axis=0, keepdims=True)
    i1 = jnp.min(jnp.where(scores == m1, idx, E), axis=0, keepdims=True)
    sel1 = idx == i1
    masked = jnp.where(sel1, -jnp.inf, scores)
    m2 = jnp.max(masked, axis=0, keepdims=True)
    i2 = jnp.min(jnp.where(masked == m2, idx, E), axis=0, keepdims=True)
    comb = jnp.where(sel1 | (idx == i2), scores, 0.0)

    er = jax.lax.broadcasted_iota(jnp.int32, (E, E * R), 1) // R
    ee = jax.lax.broadcasted_iota(jnp.int32, (E, E * R), 0)
    expand = (er == ee).astype(jnp.float32)
    comb_wide = jax.lax.dot_general(comb, expand, _DN_TL,
                                    preferred_element_type=jnp.float32)

    h = jnp.dot(xb, lfb_ref[...], preferred_element_type=jnp.float32)
    hw = (h * comb_wide * SCALING).astype(jnp.bfloat16)

    out = jax.lax.dot_general(xb, wb_ref[...], _DN_T,
                              preferred_element_type=jnp.float32)
    out += jnp.dot(hw, rfb_ref[...], preferred_element_type=jnp.float32)
    o_ref[...] = out + b_ref[...]


@jax.jit
def _run(flat, w, b2, g_t, lefts_flat, rights_flat):
    T, D = flat.shape
    grid = (T // TM,)
    return pl.pallas_call(
        _fused_kernel,
        grid=grid,
        in_specs=[
            pl.BlockSpec((TM, D), lambda i: (i, 0)),
            pl.BlockSpec((D, D), lambda i: (0, 0)),
            pl.BlockSpec((1, D), lambda i: (0, 0)),
            pl.BlockSpec((D, E), lambda i: (0, 0)),
            pl.BlockSpec((D, E * R), lambda i: (0, 0)),
            pl.BlockSpec((E * R, D), lambda i: (0, 0)),
        ],
        out_specs=pl.BlockSpec((TM, D), lambda i: (i, 0)),
        out_shape=jax.ShapeDtypeStruct((T, D), jnp.float32),
        scratch_shapes=[
            pltpu.VMEM((D, D), jnp.bfloat16),
            pltpu.VMEM((D, E * R), jnp.bfloat16),
            pltpu.VMEM((E * R, D), jnp.bfloat16),
        ],
    )(flat, w, b2, g_t, lefts_flat, rights_flat)


def kernel(hidden_states, W_lin, b_lin, gate_w, lefts, rights):
    bsz, seq_len, dim = hidden_states.shape
    flat = hidden_states.reshape(-1, dim)
    g_t = gate_w.T
    d = lefts.shape[1]
    lefts_flat = lefts.transpose(1, 0, 2).reshape(d, E * R)
    rights_flat = rights.reshape(E * R, -1)
    b2 = b_lin.reshape(1, -1)
    out = _run(flat, W_lin, b2, g_t, lefts_flat, rights_flat)
    return out.reshape(bsz, seq_len, -1)
